# baseline (speedup 1.0000x reference)
import jax
import jax.numpy as jnp
from jax import lax
from jax.experimental import pallas as pl
from jax.experimental.pallas import tpu as pltpu
from jax.experimental.pallas import tpu_sc as plsc

N = 10000
E = 320000
D = 128
G = 64

NCORE = 2
NSUB = 16
NTILE = NCORE * NSUB
CHUNK = 128
CPTD = 81
EPADD = NTILE * CPTD * CHUNK
CPTA = 79
EPADA = NTILE * CPTA * CHUNK
NA = 10240
WROWS = NA // NSUB
RB = 1000
NRB = N // RB
H3 = 3 * D


def _sc_mesh():
    return plsc.VectorSubcoreMesh(core_axis_name="c", subcore_axis_name="s")


def _deg_body(dst_hbm, z1_hbm, out_hbm, dstv, onesv, acc):
    c = lax.axis_index("c")
    s = lax.axis_index("s")
    w = c * NSUB + s
    pltpu.sync_copy(dst_hbm.at[w], dstv)
    for i in range(CHUNK // 16):
        onesv[pl.ds(i * 16, 16)] = jnp.ones((16,), jnp.float32)
    pltpu.sync_copy(z1_hbm.at[pl.ds(s * WROWS, WROWS)],
                    acc.at[pl.ds(s * WROWS, WROWS)])
    plsc.subcore_barrier()

    def body(j, carry):
        pltpu.sync_copy(onesv, acc.at[dstv.at[j]], add=True)
        return carry

    lax.fori_loop(0, CPTD, body, 0)
    plsc.subcore_barrier()
    pltpu.sync_copy(acc.at[pl.ds(s * WROWS, WROWS)],
                    out_hbm.at[c, pl.ds(s * WROWS, WROWS)])


_deg_kernel = pl.kernel(
    _deg_body,
    out_type=jax.ShapeDtypeStruct((NCORE, NA), jnp.float32),
    mesh=_sc_mesh(),
    scratch_types=[
        pltpu.VMEM((CPTD, CHUNK), jnp.int32),
        pltpu.VMEM((CHUNK,), jnp.float32),
        pltpu.VMEM_SHARED((NA,), jnp.float32),
    ],
)


def _agg_body(ma_hbm, mb_hbm, src_hbm, dst_hbm, z2_hbm, out_hbm,
              srcv, dstv, buf, acc):
    c = lax.axis_index("c")
    s = lax.axis_index("s")
    w = c * NSUB + s
    pltpu.sync_copy(src_hbm.at[w], srcv)
    pltpu.sync_copy(dst_hbm.at[w], dstv)
    pltpu.sync_copy(z2_hbm.at[pl.ds(s * WROWS, WROWS)],
                    acc.at[pl.ds(s * WROWS, WROWS)])
    plsc.subcore_barrier()

    def run(m_hbm):
        def body(j, carry):
            pltpu.sync_copy(m_hbm.at[srcv.at[j]], buf)
            pltpu.sync_copy(buf, acc.at[dstv.at[j]], add=True)
            return carry
        lax.fori_loop(0, CPTA, body, 0)

    @pl.when(c == 0)
    def _run_a():
        run(mb_hbm)

    @pl.when(c == 1)
    def _run_b():
        run(ma_hbm)

    plsc.subcore_barrier()
    pltpu.sync_copy(acc.at[pl.ds(s * WROWS, WROWS)],
                    out_hbm.at[c, pl.ds(s * WROWS, WROWS)])


_agg_kernel = pl.kernel(
    _agg_body,
    out_type=jax.ShapeDtypeStruct((NCORE, NA, D), jnp.float32),
    mesh=_sc_mesh(),
    scratch_types=[
        pltpu.VMEM((CPTA, CHUNK), jnp.int32),
        pltpu.VMEM((CPTA, CHUNK), jnp.int32),
        pltpu.VMEM((CHUNK, D), jnp.float32),
        pltpu.VMEM_SHARED((NA, D), jnp.float32),
    ],
)


def _k0_body(dp_ref, x_ref, w_ref, ma_ref, mb_ref, dinv_ref):
    deg = dp_ref[:, 0:1] + dp_ref[:, 1:2]
    dinv = jnp.where(deg > 0, lax.rsqrt(jnp.maximum(deg, 1e-12)), 0.0)
    dinv_ref[...] = dinv
    m = jnp.dot(x_ref[...] * dinv, w_ref[...],
                preferred_element_type=jnp.float32)
    ma_ref[...] = m
    mb_ref[...] = m


def _kmid_body(acc_ref, m_ref, dinv_ref, b_ref, w_ref, h_ref, ma_ref, mb_ref):
    a = acc_ref[0, :, :] + acc_ref[1, :, :] + m_ref[...]
    dinv = dinv_ref[...]
    h = jnp.maximum(a * dinv + b_ref[...], 0.0)
    h_ref[...] = h
    m = jnp.dot(h * dinv, w_ref[...],
                preferred_element_type=jnp.float32)
    ma_ref[...] = m
    mb_ref[...] = m


def _klast_body(acc_ref, m_ref, dinv_ref, b_ref, h_ref):
    a = acc_ref[0, :, :] + acc_ref[1, :, :] + m_ref[...]
    h_ref[...] = jnp.maximum(a * dinv_ref[...] + b_ref[...], 0.0)


def _pool_body(lohi_ref, b2_ref, h1_ref, h2_ref, h3_ref,
               l1w_ref, l1b_ref, l2w_ref, l2b_ref,
               o_ref, s_scr, mx_scr, cnt_scr):
    j = pl.program_id(0)

    @pl.when(j == 0)
    def _init():
        s_scr[...] = jnp.zeros((G, H3), jnp.float32)
        mx_scr[...] = jnp.full((G, H3), -jnp.inf, jnp.float32)
        cnt_scr[...] = jnp.zeros((G, 1), jnp.float32)

    bb = b2_ref[...]
    h1 = h1_ref[...]
    h2 = h2_ref[...]
    h3 = h3_ref[...]
    neg = jnp.float32(-jnp.inf)

    def seg(g, carry):
        mask = bb == g
        srow = jnp.concatenate([
            jnp.sum(jnp.where(mask, h1, 0.0), axis=0, keepdims=True),
            jnp.sum(jnp.where(mask, h2, 0.0), axis=0, keepdims=True),
            jnp.sum(jnp.where(mask, h3, 0.0), axis=0, keepdims=True),
        ], axis=1)
        mrow = jnp.concatenate([
            jnp.max(jnp.where(mask, h1, neg), axis=0, keepdims=True),
            jnp.max(jnp.where(mask, h2, neg), axis=0, keepdims=True),
            jnp.max(jnp.where(mask, h3, neg), axis=0, keepdims=True),
        ], axis=1)
        crow = jnp.sum(mask.astype(jnp.float32), axis=0, keepdims=True)
        s_scr[pl.ds(g, 1), :] = s_scr[pl.ds(g, 1), :] + srow
        mx_scr[pl.ds(g, 1), :] = jnp.maximum(mx_scr[pl.ds(g, 1), :], mrow)
        cnt_scr[pl.ds(g, 1), :] = cnt_scr[pl.ds(g, 1), :] + crow
        return carry

    lax.fori_loop(lohi_ref[j, 0], lohi_ref[j, 1] + 1, seg, 0)

    @pl.when(j == NRB - 1)
    def _fin():
        s = s_scr[...]
        cnt = cnt_scr[...]
        mean = s / jnp.maximum(cnt, 1.0)
        mx = jnp.where(cnt > 0, mx_scr[...], 0.0)
        gcat = jnp.concatenate([s, mean, mx], axis=1)
        z = jnp.maximum(
            jnp.dot(gcat, l1w_ref[...], preferred_element_type=jnp.float32)
            + l1b_ref[...], 0.0)
        o = (jnp.dot(z, l2w_ref[...], preferred_element_type=jnp.float32)
             + l2b_ref[...])
        omx = jnp.max(o, axis=1, keepdims=True)
        e = jnp.exp(o - omx)
        o_ref[...] = (o - omx) - jnp.log(jnp.sum(e, axis=1, keepdims=True))


def _row_block(j):
    return (j, 0)


def _fixed(j):
    return (0, 0)


def _fixed3(j):
    return (0, j, 0)


@jax.jit
def _impl(x, edge_index, batch, W1, b1, W2, b2, W3, b3,
          lin1_w, lin1_b, lin2_w, lin2_b):
    loop = jnp.arange(N, dtype=jnp.int32)
    npad_a = EPADA - E
    src3 = jnp.concatenate(
        [edge_index[0].astype(jnp.int32),
         jnp.zeros((npad_a,), jnp.int32)]).reshape(NTILE, CPTA, CHUNK)
    dst3 = jnp.concatenate(
        [edge_index[1].astype(jnp.int32),
         jnp.full((npad_a,), N, jnp.int32)]).reshape(NTILE, CPTA, CHUNK)
    npad_d = EPADD - E - N
    dst3d = jnp.concatenate(
        [edge_index[1].astype(jnp.int32), loop,
         jnp.full((npad_d,), N, jnp.int32)]).reshape(NTILE, CPTD, CHUNK)
    z1 = jnp.zeros((NA,), jnp.float32)
    z2 = jnp.zeros((NA, D), jnp.float32)

    degp = _deg_kernel(dst3d, z1)
    degT = degp.T[:N]

    m1a, m1b, dinv = pl.pallas_call(
        _k0_body,
        grid=(NRB,),
        in_specs=[pl.BlockSpec((RB, 2), _row_block),
                  pl.BlockSpec((RB, D), _row_block),
                  pl.BlockSpec((D, D), _fixed)],
        out_specs=[pl.BlockSpec((RB, D), _row_block),
                   pl.BlockSpec((RB, D), _row_block),
                   pl.BlockSpec((RB, 1), _row_block)],
        out_shape=[jax.ShapeDtypeStruct((N, D), jnp.float32),
                   jax.ShapeDtypeStruct((N, D), jnp.float32),
                   jax.ShapeDtypeStruct((N, 1), jnp.float32)],
    )(degT, x, W1)

    mid_call = pl.pallas_call(
        _kmid_body,
        grid=(NRB,),
        in_specs=[pl.BlockSpec((NCORE, RB, D), _fixed3),
                  pl.BlockSpec((RB, D), _row_block),
                  pl.BlockSpec((RB, 1), _row_block),
                  pl.BlockSpec((1, D), _fixed),
                  pl.BlockSpec((D, D), _fixed)],
        out_specs=[pl.BlockSpec((RB, D), _row_block),
                   pl.BlockSpec((RB, D), _row_block),
                   pl.BlockSpec((RB, D), _row_block)],
        out_shape=[jax.ShapeDtypeStruct((N, D), jnp.float32),
                   jax.ShapeDtypeStruct((N, D), jnp.float32),
                   jax.ShapeDtypeStruct((N, D), jnp.float32)],
    )

    acc1 = _agg_kernel(m1a, m1b, src3, dst3, z2)
    h1, m2a, m2b = mid_call(acc1, m1a, dinv, b1.reshape(1, D), W2)

    acc2 = _agg_kernel(m2a, m2b, src3, dst3, z2)
    h2, m3a, m3b = mid_call(acc2, m2a, dinv, b2.reshape(1, D), W3)

    acc3 = _agg_kernel(m3a, m3b, src3, dst3, z2)
    h3 = pl.pallas_call(
        _klast_body,
        grid=(NRB,),
        in_specs=[pl.BlockSpec((NCORE, RB, D), _fixed3),
                  pl.BlockSpec((RB, D), _row_block),
                  pl.BlockSpec((RB, 1), _row_block),
                  pl.BlockSpec((1, D), _fixed)],
        out_specs=pl.BlockSpec((RB, D), _row_block),
        out_shape=jax.ShapeDtypeStruct((N, D), jnp.float32),
    )(acc3, m3a, dinv, b3.reshape(1, D))

    bi = batch.astype(jnp.int32)
    batch2 = bi.reshape(N, 1)
    br = bi.reshape(NRB, RB)
    lohi = jnp.stack([br[:, 0], br[:, -1]], axis=1)

    out = pl.pallas_call(
        _pool_body,
        grid=(NRB,),
        in_specs=[pl.BlockSpec(memory_space=pltpu.SMEM),
                  pl.BlockSpec((RB, 1), _row_block),
                  pl.BlockSpec((RB, D), _row_block),
                  pl.BlockSpec((RB, D), _row_block),
                  pl.BlockSpec((RB, D), _row_block),
                  pl.BlockSpec((3 * H3, D), _fixed),
                  pl.BlockSpec((1, D), _fixed),
                  pl.BlockSpec((D, 2), _fixed),
                  pl.BlockSpec((1, 2), _fixed)],
        out_specs=pl.BlockSpec((G, 2), _fixed),
        out_shape=jax.ShapeDtypeStruct((G, 2), jnp.float32),
        scratch_shapes=[pltpu.VMEM((G, H3), jnp.float32),
                        pltpu.VMEM((G, H3), jnp.float32),
                        pltpu.VMEM((G, 1), jnp.float32)],
    )(lohi, batch2, h1, h2, h3,
      lin1_w, lin1_b.reshape(1, D), lin2_w, lin2_b.reshape(1, 2))
    return out


def kernel(x, edge_index, batch, W1, b1, W2, b2, W3, b3,
           lin1_w, lin1_b, lin2_w, lin2_b):
    return _impl(x, edge_index, batch, W1, b1, W2, b2, W3, b3,
                 lin1_w, lin1_b, lin2_w, lin2_b)


# speedup vs baseline: 1.3595x; 1.3595x over previous
import jax
import jax.numpy as jnp
from jax import lax
from jax.experimental import pallas as pl
from jax.experimental.pallas import tpu as pltpu
from jax.experimental.pallas import tpu_sc as plsc

N = 10000
E = 320000
D = 128
G = 64

NCORE = 2
NSUB = 16
NTILE = NCORE * NSUB
CHUNK = 128
CPT0 = 81
CPT1 = 81
CPTM = max(CPT0, CPT1)
EPAD = NSUB * (CPT0 + CPT1) * CHUNK
NA = 10240
WROWS = NA // NSUB
RB = 1000
NRB = N // RB
H3 = 3 * D


def _sc_mesh():
    return plsc.VectorSubcoreMesh(core_axis_name="c", subcore_axis_name="s")


def _deg_body(dst_hbm, z1_hbm, out_hbm, dstv, onesv, acc):
    c = lax.axis_index("c")
    s = lax.axis_index("s")
    w = c * NSUB + s
    nchunk = jnp.where(c == 0, CPT0, CPT1)
    pltpu.sync_copy(dst_hbm.at[w], dstv)
    for i in range(CHUNK // 16):
        onesv[pl.ds(i * 16, 16)] = jnp.ones((16,), jnp.float32)
    pltpu.sync_copy(z1_hbm.at[pl.ds(s * WROWS, WROWS)],
                    acc.at[pl.ds(s * WROWS, WROWS)])
    plsc.subcore_barrier()

    def body(j, carry):
        pltpu.sync_copy(onesv, acc.at[dstv.at[j]], add=True)
        return carry

    lax.fori_loop(0, nchunk, body, 0)
    plsc.subcore_barrier()
    pltpu.sync_copy(acc.at[pl.ds(s * WROWS, WROWS)],
                    out_hbm.at[c, pl.ds(s * WROWS, WROWS)])


_deg_kernel = pl.kernel(
    _deg_body,
    out_type=jax.ShapeDtypeStruct((NCORE, NA), jnp.float32),
    mesh=_sc_mesh(),
    scratch_types=[
        pltpu.VMEM((CPTM, CHUNK), jnp.int32),
        pltpu.VMEM((CHUNK,), jnp.float32),
        pltpu.VMEM_SHARED((NA,), jnp.float32),
    ],
)


def _agg_body(ma_hbm, mb_hbm, src_hbm, dst_hbm, z2_hbm, out_hbm,
              srcv, dstv, buf, acc):
    c = lax.axis_index("c")
    s = lax.axis_index("s")
    w = c * NSUB + s
    nchunk = jnp.where(c == 0, CPT0, CPT1)
    pltpu.sync_copy(src_hbm.at[w], srcv)
    pltpu.sync_copy(dst_hbm.at[w], dstv)
    pltpu.sync_copy(z2_hbm.at[pl.ds(s * WROWS, WROWS)],
                    acc.at[pl.ds(s * WROWS, WROWS)])
    plsc.subcore_barrier()

    def run(m_hbm):
        def body(j, carry):
            pltpu.sync_copy(m_hbm.at[srcv.at[j]], buf)
            pltpu.sync_copy(buf, acc.at[dstv.at[j]], add=True)
            return carry
        lax.fori_loop(0, nchunk, body, 0)

    @pl.when(c == 0)
    def _run_a():
        run(mb_hbm)

    @pl.when(c == 1)
    def _run_b():
        run(ma_hbm)

    plsc.subcore_barrier()
    pltpu.sync_copy(acc.at[pl.ds(s * WROWS, WROWS)],
                    out_hbm.at[c, pl.ds(s * WROWS, WROWS)])


_agg_kernel = pl.kernel(
    _agg_body,
    out_type=jax.ShapeDtypeStruct((NCORE, NA, D), jnp.float32),
    mesh=_sc_mesh(),
    scratch_types=[
        pltpu.VMEM((CPTM, CHUNK), jnp.int32),
        pltpu.VMEM((CPTM, CHUNK), jnp.int32),
        pltpu.VMEM((CHUNK, D), jnp.float32),
        pltpu.VMEM_SHARED((NA, D), jnp.float32),
    ],
)


def _k0_body(dp_ref, x_ref, w_ref, ma_ref, mb_ref, dinv_ref):
    deg = dp_ref[:, 0:1] + dp_ref[:, 1:2]
    dinv = jnp.where(deg > 0, lax.rsqrt(jnp.maximum(deg, 1e-12)), 0.0)
    dinv_ref[...] = dinv
    m = jnp.dot(x_ref[...] * dinv, w_ref[...],
                preferred_element_type=jnp.float32)
    ma_ref[...] = m
    mb_ref[...] = m


def _kmid_body(acc_ref, dinv_ref, b_ref, w_ref, h_ref, ma_ref, mb_ref):
    a = acc_ref[0, :, :] + acc_ref[1, :, :]
    dinv = dinv_ref[...]
    h = jnp.maximum(a * dinv + b_ref[...], 0.0)
    h_ref[...] = h
    m = jnp.dot(h * dinv, w_ref[...],
                preferred_element_type=jnp.float32)
    ma_ref[...] = m
    mb_ref[...] = m


def _klast_body(acc_ref, dinv_ref, b_ref, h_ref):
    a = acc_ref[0, :, :] + acc_ref[1, :, :]
    h_ref[...] = jnp.maximum(a * dinv_ref[...] + b_ref[...], 0.0)


def _pool_body(lohi_ref, b2_ref, h1_ref, h2_ref, h3_ref,
               l1w_ref, l1b_ref, l2w_ref, l2b_ref,
               o_ref, s_scr, mx_scr, cnt_scr):
    j = pl.program_id(0)

    @pl.when(j == 0)
    def _init():
        s_scr[...] = jnp.zeros((G, H3), jnp.float32)
        mx_scr[...] = jnp.full((G, H3), -jnp.inf, jnp.float32)
        cnt_scr[...] = jnp.zeros((G, 1), jnp.float32)

    bb = b2_ref[...]
    h1 = h1_ref[...]
    h2 = h2_ref[...]
    h3 = h3_ref[...]
    neg = jnp.float32(-jnp.inf)

    def seg(g, carry):
        mask = bb == g
        srow = jnp.concatenate([
            jnp.sum(jnp.where(mask, h1, 0.0), axis=0, keepdims=True),
            jnp.sum(jnp.where(mask, h2, 0.0), axis=0, keepdims=True),
            jnp.sum(jnp.where(mask, h3, 0.0), axis=0, keepdims=True),
        ], axis=1)
        mrow = jnp.concatenate([
            jnp.max(jnp.where(mask, h1, neg), axis=0, keepdims=True),
            jnp.max(jnp.where(mask, h2, neg), axis=0, keepdims=True),
            jnp.max(jnp.where(mask, h3, neg), axis=0, keepdims=True),
        ], axis=1)
        crow = jnp.sum(mask.astype(jnp.float32), axis=0, keepdims=True)
        s_scr[pl.ds(g, 1), :] = s_scr[pl.ds(g, 1), :] + srow
        mx_scr[pl.ds(g, 1), :] = jnp.maximum(mx_scr[pl.ds(g, 1), :], mrow)
        cnt_scr[pl.ds(g, 1), :] = cnt_scr[pl.ds(g, 1), :] + crow
        return carry

    lax.fori_loop(lohi_ref[j, 0], lohi_ref[j, 1] + 1, seg, 0)

    @pl.when(j == NRB - 1)
    def _fin():
        s = s_scr[...]
        cnt = cnt_scr[...]
        mean = s / jnp.maximum(cnt, 1.0)
        mx = jnp.where(cnt > 0, mx_scr[...], 0.0)
        gcat = jnp.concatenate([s, mean, mx], axis=1)
        z = jnp.maximum(
            jnp.dot(gcat, l1w_ref[...], preferred_element_type=jnp.float32)
            + l1b_ref[...], 0.0)
        o = (jnp.dot(z, l2w_ref[...], preferred_element_type=jnp.float32)
             + l2b_ref[...])
        omx = jnp.max(o, axis=1, keepdims=True)
        e = jnp.exp(o - omx)
        o_ref[...] = (o - omx) - jnp.log(jnp.sum(e, axis=1, keepdims=True))


def _row_block(j):
    return (j, 0)


def _fixed(j):
    return (0, 0)


def _fixed3(j):
    return (0, j, 0)


@jax.jit
def _impl(x, edge_index, batch, W1, b1, W2, b2, W3, b3,
          lin1_w, lin1_b, lin2_w, lin2_b):
    loop = jnp.arange(N, dtype=jnp.int32)
    npad = EPAD - E - N
    src = jnp.concatenate([edge_index[0].astype(jnp.int32), loop,
                           jnp.zeros((npad,), jnp.int32)])
    dst = jnp.concatenate([edge_index[1].astype(jnp.int32), loop,
                           jnp.full((npad,), N, jnp.int32)])
    total0 = NSUB * CPT0 * CHUNK

    def _split(a, fill):
        a0 = a[:total0].reshape(NSUB, CPT0, CHUNK)
        a1 = a[total0:].reshape(NSUB, CPT1, CHUNK)

        def _padc(p, cpt):
            if cpt == CPTM:
                return p
            padp = jnp.full((NSUB, CPTM - cpt, CHUNK), fill, jnp.int32)
            return jnp.concatenate([p, padp], axis=1)

        return jnp.concatenate([_padc(a0, CPT0), _padc(a1, CPT1)], axis=0)

    src3 = _split(src, 0)
    dst3 = _split(dst, N)
    z1 = jnp.zeros((NA,), jnp.float32)
    z2 = jnp.zeros((NA, D), jnp.float32)

    degp = _deg_kernel(dst3, z1)
    degT = degp.T[:N]

    m1a, m1b, dinv = pl.pallas_call(
        _k0_body,
        grid=(NRB,),
        in_specs=[pl.BlockSpec((RB, 2), _row_block),
                  pl.BlockSpec((RB, D), _row_block),
                  pl.BlockSpec((D, D), _fixed)],
        out_specs=[pl.BlockSpec((RB, D), _row_block),
                   pl.BlockSpec((RB, D), _row_block),
                   pl.BlockSpec((RB, 1), _row_block)],
        out_shape=[jax.ShapeDtypeStruct((N, D), jnp.float32),
                   jax.ShapeDtypeStruct((N, D), jnp.float32),
                   jax.ShapeDtypeStruct((N, 1), jnp.float32)],
    )(degT, x, W1)

    mid_call = pl.pallas_call(
        _kmid_body,
        grid=(NRB,),
        in_specs=[pl.BlockSpec((NCORE, RB, D), _fixed3),
                  pl.BlockSpec((RB, 1), _row_block),
                  pl.BlockSpec((1, D), _fixed),
                  pl.BlockSpec((D, D), _fixed)],
        out_specs=[pl.BlockSpec((RB, D), _row_block),
                   pl.BlockSpec((RB, D), _row_block),
                   pl.BlockSpec((RB, D), _row_block)],
        out_shape=[jax.ShapeDtypeStruct((N, D), jnp.float32),
                   jax.ShapeDtypeStruct((N, D), jnp.float32),
                   jax.ShapeDtypeStruct((N, D), jnp.float32)],
    )

    acc1 = _agg_kernel(m1a, m1b, src3, dst3, z2)
    h1, m2a, m2b = mid_call(acc1, dinv, b1.reshape(1, D), W2)

    acc2 = _agg_kernel(m2a, m2b, src3, dst3, z2)
    h2, m3a, m3b = mid_call(acc2, dinv, b2.reshape(1, D), W3)

    acc3 = _agg_kernel(m3a, m3b, src3, dst3, z2)
    h3 = pl.pallas_call(
        _klast_body,
        grid=(NRB,),
        in_specs=[pl.BlockSpec((NCORE, RB, D), _fixed3),
                  pl.BlockSpec((RB, 1), _row_block),
                  pl.BlockSpec((1, D), _fixed)],
        out_specs=pl.BlockSpec((RB, D), _row_block),
        out_shape=jax.ShapeDtypeStruct((N, D), jnp.float32),
    )(acc3, dinv, b3.reshape(1, D))

    bi = batch.astype(jnp.int32)
    batch2 = bi.reshape(N, 1)
    br = bi.reshape(NRB, RB)
    lohi = jnp.stack([br[:, 0], br[:, -1]], axis=1)

    out = pl.pallas_call(
        _pool_body,
        grid=(NRB,),
        in_specs=[pl.BlockSpec(memory_space=pltpu.SMEM),
                  pl.BlockSpec((RB, 1), _row_block),
                  pl.BlockSpec((RB, D), _row_block),
                  pl.BlockSpec((RB, D), _row_block),
                  pl.BlockSpec((RB, D), _row_block),
                  pl.BlockSpec((3 * H3, D), _fixed),
                  pl.BlockSpec((1, D), _fixed),
                  pl.BlockSpec((D, 2), _fixed),
                  pl.BlockSpec((1, 2), _fixed)],
        out_specs=pl.BlockSpec((G, 2), _fixed),
        out_shape=jax.ShapeDtypeStruct((G, 2), jnp.float32),
        scratch_shapes=[pltpu.VMEM((G, H3), jnp.float32),
                        pltpu.VMEM((G, H3), jnp.float32),
                        pltpu.VMEM((G, 1), jnp.float32)],
    )(lohi, batch2, h1, h2, h3,
      lin1_w, lin1_b.reshape(1, D), lin2_w, lin2_b.reshape(1, 2))
    return out


def kernel(x, edge_index, batch, W1, b1, W2, b2, W3, b3,
           lin1_w, lin1_b, lin2_w, lin2_b):
    return _impl(x, edge_index, batch, W1, b1, W2, b2, W3, b3,
                 lin1_w, lin1_b, lin2_w, lin2_b)
